# baseline (speedup 1.0000x reference)
import functools

import jax
import jax.numpy as jnp
from jax import lax
from jax.experimental import pallas as pl
from jax.experimental.pallas import tpu as pltpu
from jax.experimental.pallas import tpu_sc as plsc

B, T, D = 2, 4096, 4096
_TT = 1024


def _score_body(x_ref, w_ref, s_ref, p_ref):
    xb = x_ref[0]
    w = w_ref[...]
    s = jax.lax.dot_general(
        xb, w,
        (((1,), (0,)), ((), ())),
        precision=jax.lax.Precision.DEFAULT,
        preferred_element_type=jnp.float32,
    )
    s_ref[0] = s
    p_ref[0] = jax.nn.sigmoid(s)


def _scores_probs(x, w):
    nt = (B * T) // _TT
    x3 = x.reshape(nt, _TT, D)
    outs = pl.pallas_call(
        _score_body,
        grid=(nt,),
        in_specs=[
            pl.BlockSpec((1, _TT, D), lambda i: (i, 0, 0)),
            pl.BlockSpec((D, 1), lambda i: (0, 0)),
        ],
        out_specs=[
            pl.BlockSpec((1, _TT, 1), lambda i: (i, 0, 0)),
            pl.BlockSpec((1, _TT, 1), lambda i: (i, 0, 0)),
        ],
        out_shape=[
            jax.ShapeDtypeStruct((nt, _TT, 1), jnp.float32),
            jax.ShapeDtypeStruct((nt, _TT, 1), jnp.float32),
        ],
        compiler_params=pltpu.CompilerParams(
            vmem_limit_bytes=100 * 1024 * 1024),
    )(x3, w.reshape(D, 1))
    return outs[0].reshape(B, T), outs[1].reshape(B, T)


def _mask_body(cap_ref, p_ref, m_ref):
    cap = cap_ref[0, 0]
    p = p_ref[...]
    keys = jax.lax.bitcast_convert_type(p, jnp.int32)
    capv = jnp.full((B, 1), cap, jnp.int32)

    def bis_step(_, lohi):
        lo, hi = lohi
        mid = lo + ((hi - lo + 1) >> 1)
        cnt = jnp.sum((keys >= mid).astype(jnp.int32), axis=-1, keepdims=True)
        ok = cnt >= capv
        return jnp.where(ok, mid, lo), jnp.where(ok, hi, mid - 1)

    lo0 = jnp.zeros((B, 1), jnp.int32)
    hi0 = jnp.full((B, 1), 0x3F800000, jnp.int32)
    kstar, _ = jax.lax.fori_loop(0, 30, bis_step, (lo0, hi0))

    gt = keys > kstar
    eq = keys == kstar
    g = jnp.sum(gt.astype(jnp.int32), axis=-1, keepdims=True)
    rem = capv - g
    eqi = eq.astype(jnp.int32)
    iota = jax.lax.broadcasted_iota(jnp.int32, (B, T), 1)

    def idx_step(_, lohi):
        lo, hi = lohi
        mid = lo + ((hi - lo + 1) >> 1)
        cnt = jnp.sum(jnp.where(iota < mid, eqi, 0), axis=-1, keepdims=True)
        ok = cnt <= rem
        return jnp.where(ok, mid, lo), jnp.where(ok, hi, mid - 1)

    lo0i = jnp.zeros((B, 1), jnp.int32)
    hi0i = jnp.full((B, 1), T, jnp.int32)
    cstar, _ = jax.lax.fori_loop(0, 13, idx_step, (lo0i, hi0i))

    m_ref[...] = (gt | (eq & (iota < cstar))).astype(jnp.float32)


def _mask(probs, cap2d):
    return pl.pallas_call(
        _mask_body,
        in_specs=[
            pl.BlockSpec((1, 1), lambda: (0, 0)),
            pl.BlockSpec((B, T), lambda: (0, 0)),
        ],
        out_specs=pl.BlockSpec((B, T), lambda: (0, 0)),
        out_shape=jax.ShapeDtypeStruct((B, T), jnp.float32),
    )(cap2d, probs)


_L = 16
_NV = T // _L


def _mask_sc(probs, capv):
    mesh = plsc.VectorSubcoreMesh(core_axis_name="c", subcore_axis_name="s")

    @functools.partial(
        pl.kernel,
        out_type=jax.ShapeDtypeStruct((B * T,), jnp.float32),
        mesh=mesh,
        scratch_types=[
            pltpu.VMEM((T,), jnp.float32),
            pltpu.VMEM((T,), jnp.int32),
            pltpu.VMEM((T,), jnp.float32),
            pltpu.VMEM((_L,), jnp.int32),
        ],
    )
    def k(probs_hbm, cap_hbm, out_hbm, pv, kv, mv, capv_ref):
        row = lax.axis_index("c")
        sub = lax.axis_index("s")

        @pl.when(sub == 0)
        def _():
            pltpu.sync_copy(probs_hbm.at[pl.ds(row * T, T)], pv)
            pltpu.sync_copy(cap_hbm, capv_ref)
            capk = capv_ref[...]
            def mk(i, carry):
                kv[pl.ds(i * _L, _L)] = lax.bitcast_convert_type(
                    pv[pl.ds(i * _L, _L)], jnp.int32)
                return carry
            lax.fori_loop(0, _NV, mk, 0)

            lanes = lax.iota(jnp.int32, _L)

            def splat_total(v):
                for stride in (8, 4, 2, 1):
                    v = v + v.at[lanes ^ stride].get(
                        mode="promise_in_bounds")
                return v

            def count_ge(thr):
                def body(j, acc):
                    kk = kv[pl.ds(j * _L, _L)]
                    return acc + jnp.where(kk >= thr, 1, 0)
                return lax.fori_loop(0, _NV, body,
                                     jnp.zeros((_L,), jnp.int32))

            def bstep(_, lh):
                lo, hi = lh
                mid = lo + ((hi - lo + 1) >> 1)
                ok = splat_total(count_ge(mid)) >= capk
                return (jnp.where(ok, mid, lo), jnp.where(ok, hi, mid - 1))

            lo0 = jnp.zeros((_L,), jnp.int32)
            hi0 = jnp.full((_L,), 0x3F800000, jnp.int32)
            kstar, _ = lax.fori_loop(0, 30, bstep, (lo0, hi0))

            rem = capk - splat_total(count_ge(kstar + 1))

            def wstep(i, carry):
                kk = kv[pl.ds(i * _L, _L)]
                gt = kk > kstar
                eq = kk == kstar
                eqi = jnp.where(eq, 1, 0)
                csum = eqi
                for stride in (1, 2, 4, 8):
                    shifted = csum.at[jnp.maximum(lanes - stride, 0)].get(
                        mode="promise_in_bounds")
                    csum = csum + jnp.where(lanes >= stride, shifted, 0)
                pref = carry + csum - eqi
                take = jnp.logical_or(gt, jnp.logical_and(eq, pref < rem))
                mv[pl.ds(i * _L, _L)] = jnp.where(take, 1.0, 0.0)
                return carry + splat_total(eqi)
            lax.fori_loop(0, _NV, wstep, jnp.zeros((_L,), jnp.int32))

            pltpu.sync_copy(mv, out_hbm.at[pl.ds(row * T, T)])

    return k(probs.reshape(B * T), capv).reshape(B, T)


def _zloss_body(s_ref, z_ref):
    s = s_ref[...]
    m = jnp.max(s, axis=-1, keepdims=True)
    lse = m + jnp.log(jnp.sum(jnp.exp(s - m), axis=-1, keepdims=True))
    z_ref[...] = jnp.mean(lse * lse).reshape(1, 1)


def _zloss(scores):
    return pl.pallas_call(
        _zloss_body,
        in_specs=[pl.BlockSpec((B, T), lambda: (0, 0))],
        out_specs=pl.BlockSpec((1, 1), lambda: (0, 0)),
        out_shape=jax.ShapeDtypeStruct((1, 1), jnp.float32),
    )(scores)


def kernel(x, capacity, W):
    scores, probs = _scores_probs(x, W)
    capv = jnp.full((_L,), capacity, jnp.int32)
    mask = _mask_sc(probs, capv)
    z = _zloss(scores)
    return (mask, probs, z[0, 0])


# speedup vs baseline: 1.2706x; 1.2706x over previous
import functools

import jax
import jax.numpy as jnp
from jax import lax
from jax.experimental import pallas as pl
from jax.experimental.pallas import tpu as pltpu
from jax.experimental.pallas import tpu_sc as plsc

B, T, D = 2, 4096, 4096
_TT = 1024


def _score_body(x_ref, w_ref, s_ref, p_ref):
    xb = x_ref[0]
    w = w_ref[...]
    s = jax.lax.dot_general(
        xb, w,
        (((1,), (0,)), ((), ())),
        precision=jax.lax.Precision.DEFAULT,
        preferred_element_type=jnp.float32,
    )
    s_ref[0] = s
    p_ref[0] = jax.nn.sigmoid(s)


def _scores_probs(x, w):
    nt = (B * T) // _TT
    x3 = x.reshape(nt, _TT, D)
    outs = pl.pallas_call(
        _score_body,
        grid=(nt,),
        in_specs=[
            pl.BlockSpec((1, _TT, D), lambda i: (i, 0, 0)),
            pl.BlockSpec((D, 1), lambda i: (0, 0)),
        ],
        out_specs=[
            pl.BlockSpec((1, _TT, 1), lambda i: (i, 0, 0)),
            pl.BlockSpec((1, _TT, 1), lambda i: (i, 0, 0)),
        ],
        out_shape=[
            jax.ShapeDtypeStruct((nt, _TT, 1), jnp.float32),
            jax.ShapeDtypeStruct((nt, _TT, 1), jnp.float32),
        ],
        compiler_params=pltpu.CompilerParams(
            vmem_limit_bytes=100 * 1024 * 1024),
    )(x3, w.reshape(D, 1))
    return outs[0].reshape(B, T), outs[1].reshape(B, T)


def _mask_body(cap_ref, p_ref, m_ref):
    cap = cap_ref[0, 0]
    p = p_ref[...]
    keys = jax.lax.bitcast_convert_type(p, jnp.int32)
    capv = jnp.full((B, 1), cap, jnp.int32)

    def bis_step(_, lohi):
        lo, hi = lohi
        mid = lo + ((hi - lo + 1) >> 1)
        cnt = jnp.sum((keys >= mid).astype(jnp.int32), axis=-1, keepdims=True)
        ok = cnt >= capv
        return jnp.where(ok, mid, lo), jnp.where(ok, hi, mid - 1)

    lo0 = jnp.zeros((B, 1), jnp.int32)
    hi0 = jnp.full((B, 1), 0x3F800000, jnp.int32)
    kstar, _ = jax.lax.fori_loop(0, 30, bis_step, (lo0, hi0))

    gt = keys > kstar
    eq = keys == kstar
    g = jnp.sum(gt.astype(jnp.int32), axis=-1, keepdims=True)
    rem = capv - g
    eqi = eq.astype(jnp.int32)
    iota = jax.lax.broadcasted_iota(jnp.int32, (B, T), 1)

    def idx_step(_, lohi):
        lo, hi = lohi
        mid = lo + ((hi - lo + 1) >> 1)
        cnt = jnp.sum(jnp.where(iota < mid, eqi, 0), axis=-1, keepdims=True)
        ok = cnt <= rem
        return jnp.where(ok, mid, lo), jnp.where(ok, hi, mid - 1)

    lo0i = jnp.zeros((B, 1), jnp.int32)
    hi0i = jnp.full((B, 1), T, jnp.int32)
    cstar, _ = jax.lax.fori_loop(0, 13, idx_step, (lo0i, hi0i))

    m_ref[...] = (gt | (eq & (iota < cstar))).astype(jnp.float32)


def _mask(probs, cap2d):
    return pl.pallas_call(
        _mask_body,
        in_specs=[
            pl.BlockSpec((1, 1), lambda: (0, 0)),
            pl.BlockSpec((B, T), lambda: (0, 0)),
        ],
        out_specs=pl.BlockSpec((B, T), lambda: (0, 0)),
        out_shape=jax.ShapeDtypeStruct((B, T), jnp.float32),
    )(cap2d, probs)


_L = 16
_NV = T // _L


def _mask_sc(probs, capv):
    mesh = plsc.VectorSubcoreMesh(core_axis_name="c", subcore_axis_name="s")

    @functools.partial(
        pl.kernel,
        out_type=jax.ShapeDtypeStruct((B * T,), jnp.float32),
        mesh=mesh,
        scratch_types=[
            pltpu.VMEM((T,), jnp.float32),
            pltpu.VMEM((T,), jnp.int32),
            pltpu.VMEM((T,), jnp.float32),
            pltpu.VMEM((_L,), jnp.int32),
        ],
    )
    def k(probs_hbm, cap_hbm, out_hbm, pv, kv, mv, capv_ref):
        row = lax.axis_index("c")
        sub = lax.axis_index("s")

        @pl.when(sub == 0)
        def _():
            pltpu.sync_copy(probs_hbm.at[pl.ds(row * T, T)], pv)
            pltpu.sync_copy(cap_hbm, capv_ref)
            capk = capv_ref[...]
            def mk(i, carry):
                kv[pl.ds(i * _L, _L)] = lax.bitcast_convert_type(
                    pv[pl.ds(i * _L, _L)], jnp.int32)
                return carry
            lax.fori_loop(0, _NV, mk, 0)

            lanes = lax.iota(jnp.int32, _L)

            def splat_total(v):
                for stride in (8, 4, 2, 1):
                    v = v + v.at[lanes ^ stride].get(
                        mode="promise_in_bounds")
                return v

            def count_ge(thr):
                def body(j, acc):
                    for u in range(8):
                        kk = kv[pl.ds((j * 8 + u) * _L, _L)]
                        acc = acc + jnp.where(kk >= thr, 1, 0)
                    return acc
                return lax.fori_loop(0, _NV // 8, body,
                                     jnp.zeros((_L,), jnp.int32))

            def bstep(_, lh):
                lo, hi = lh
                mid = lo + ((hi - lo + 1) >> 1)
                ok = splat_total(count_ge(mid)) >= capk
                return (jnp.where(ok, mid, lo), jnp.where(ok, hi, mid - 1))

            lo0 = jnp.zeros((_L,), jnp.int32)
            hi0 = jnp.full((_L,), 0x3F800000, jnp.int32)
            kstar, _ = lax.fori_loop(0, 30, bstep, (lo0, hi0))

            rem = capk - splat_total(count_ge(kstar + 1))

            def wstep(i, carry):
                kk = kv[pl.ds(i * _L, _L)]
                gt = kk > kstar
                eq = kk == kstar
                eqi = jnp.where(eq, 1, 0)
                csum = eqi
                for stride in (1, 2, 4, 8):
                    shifted = csum.at[jnp.maximum(lanes - stride, 0)].get(
                        mode="promise_in_bounds")
                    csum = csum + jnp.where(lanes >= stride, shifted, 0)
                pref = carry + csum - eqi
                take = jnp.logical_or(gt, jnp.logical_and(eq, pref < rem))
                mv[pl.ds(i * _L, _L)] = jnp.where(take, 1.0, 0.0)
                return carry + splat_total(eqi)
            lax.fori_loop(0, _NV, wstep, jnp.zeros((_L,), jnp.int32))

            pltpu.sync_copy(mv, out_hbm.at[pl.ds(row * T, T)])

    return k(probs.reshape(B * T), capv).reshape(B, T)


def _zloss_body(s_ref, z_ref):
    s = s_ref[...]
    m = jnp.max(s, axis=-1, keepdims=True)
    lse = m + jnp.log(jnp.sum(jnp.exp(s - m), axis=-1, keepdims=True))
    z_ref[...] = jnp.mean(lse * lse).reshape(1, 1)


def _zloss(scores):
    return pl.pallas_call(
        _zloss_body,
        in_specs=[pl.BlockSpec((B, T), lambda: (0, 0))],
        out_specs=pl.BlockSpec((1, 1), lambda: (0, 0)),
        out_shape=jax.ShapeDtypeStruct((1, 1), jnp.float32),
    )(scores)


def kernel(x, capacity, W):
    scores, probs = _scores_probs(x, W)
    capv = jnp.full((_L,), capacity, jnp.int32)
    mask = _mask_sc(probs, capv)
    z = _zloss(scores)
    return (mask, probs, z[0, 0])
